# baseline (device time: 189817 ns/iter reference)
import jax
import jax.numpy as jnp
from jax import lax
from jax.experimental import pallas as pl
from jax.experimental.pallas import tpu as pltpu

T = 2048
TS = 1024
D = 1024
F = 4096
E = 16
EL = 8
ELY = 4
K = 2
C2 = 168
FT = 1024


def _rag_body(r_ref, rt_ref, send_sem, recv_sem):
    my_x = lax.axis_index("x")
    my_y = lax.axis_index("y")
    peer = (1 - my_x, my_y)

    barrier = pltpu.get_barrier_semaphore()
    pl.semaphore_signal(
        barrier, inc=1, device_id=peer, device_id_type=pl.DeviceIdType.MESH
    )
    pl.semaphore_wait(barrier, 1)

    rt_ref[pl.ds(my_x * EL, EL), :] = r_ref[...]
    rdma = pltpu.make_async_remote_copy(
        src_ref=r_ref,
        dst_ref=rt_ref.at[pl.ds(my_x * EL, EL), :],
        send_sem=send_sem,
        recv_sem=recv_sem,
        device_id=peer,
        device_id_type=pl.DeviceIdType.MESH,
    )
    rdma.start()
    rdma.wait()


def _router_ag(rt_shard):
    return pl.pallas_call(
        _rag_body,
        out_shape=jax.ShapeDtypeStruct((E, D), jnp.float32),
        in_specs=[pl.BlockSpec(memory_space=pltpu.VMEM)],
        out_specs=pl.BlockSpec(memory_space=pltpu.VMEM),
        scratch_shapes=[pltpu.SemaphoreType.DMA, pltpu.SemaphoreType.DMA],
        compiler_params=pltpu.CompilerParams(collective_id=0),
    )(rt_shard)


def _meta_body(m_ref, out_ref, send_sem, recv_sem):
    my_x = lax.axis_index("x")
    my_y = lax.axis_index("y")
    peer = (1 - my_x, my_y)

    barrier = pltpu.get_barrier_semaphore()
    pl.semaphore_signal(
        barrier, inc=1, device_id=peer, device_id_type=pl.DeviceIdType.MESH
    )
    pl.semaphore_wait(barrier, 1)

    rdma = pltpu.make_async_remote_copy(
        src_ref=m_ref,
        dst_ref=out_ref,
        send_sem=send_sem,
        recv_sem=recv_sem,
        device_id=peer,
        device_id_type=pl.DeviceIdType.MESH,
    )
    rdma.start()
    rdma.wait()


def _meta_exchange(meta_mine):
    return pl.pallas_call(
        _meta_body,
        out_shape=jax.ShapeDtypeStruct((TS, 8), jnp.float32),
        in_specs=[pl.BlockSpec(memory_space=pltpu.VMEM)],
        out_specs=pl.BlockSpec(memory_space=pltpu.VMEM),
        scratch_shapes=[pltpu.SemaphoreType.DMA, pltpu.SemaphoreType.DMA],
        compiler_params=pltpu.CompilerParams(collective_id=2),
    )(meta_mine)


NF = F // FT


def _mega_body(
    eids_ref, x_ref, down_ref, selp_ref, w1_ref, w2_ref, y_ref,
    xpeer_buf, dpeer, send_sem, recv_sem,
):
    s = pl.program_id(0)
    e = pl.program_id(1)
    f = pl.program_id(2)
    my_x = lax.axis_index("x")
    my_y = lax.axis_index("y")
    peer = (1 - my_x, my_y)

    rdma = pltpu.make_async_remote_copy(
        src_ref=x_ref,
        dst_ref=xpeer_buf,
        send_sem=send_sem,
        recv_sem=recv_sem,
        device_id=peer,
        device_id_type=pl.DeviceIdType.MESH,
    )

    @pl.when((s == 0) & (e == 0) & (f == 0))
    def _():
        barrier = pltpu.get_barrier_semaphore()
        pl.semaphore_signal(
            barrier, inc=1, device_id=peer,
            device_id_type=pl.DeviceIdType.MESH,
        )
        pl.semaphore_wait(barrier, 1)
        rdma.start()

    @pl.when((s == 1) & (e == 0) & (f == 0))
    def _():
        rdma.wait_recv()

    @pl.when((s == 1) & (f == 0))
    def _():
        dpeer[...] = jnp.dot(
            selp_ref[0], xpeer_buf[...], preferred_element_type=jnp.float32
        )

    rows = jnp.where(s == 0, down_ref[0], dpeer[...])
    h = jnp.maximum(
        jnp.dot(rows, w1_ref[0], preferred_element_type=jnp.float32), 0.0
    )
    contrib = jnp.dot(h, w2_ref[0], preferred_element_type=jnp.float32)

    @pl.when(f == 0)
    def _():
        y_ref[0, 0, :, :] = contrib

    @pl.when(f != 0)
    def _():
        y_ref[0, 0, :, :] += contrib

    @pl.when((s == 1) & (e == ELY - 1) & (f == NF - 1))
    def _():
        rdma.wait_send()


def _mega_ffn(eids, x, dbuf_own, selp, W1, W2):
    grid_spec = pltpu.PrefetchScalarGridSpec(
        num_scalar_prefetch=1,
        grid=(2, ELY, NF),
        in_specs=[
            pl.BlockSpec((TS, D), lambda s, e, f, eids: (0, 0)),
            pl.BlockSpec((1, C2, D), lambda s, e, f, eids: (e, 0, 0)),
            pl.BlockSpec((1, C2, TS), lambda s, e, f, eids: (e, 0, 0)),
            pl.BlockSpec((1, D, FT), lambda s, e, f, eids: (eids[e], 0, f)),
            pl.BlockSpec((1, FT, D), lambda s, e, f, eids: (eids[e], f, 0)),
        ],
        out_specs=pl.BlockSpec(
            (1, 1, C2, D), lambda s, e, f, eids: (e, s, 0, 0)
        ),
        scratch_shapes=[
            pltpu.VMEM((TS, D), jnp.float32),
            pltpu.VMEM((C2, D), jnp.float32),
            pltpu.SemaphoreType.DMA,
            pltpu.SemaphoreType.DMA,
        ],
    )
    return pl.pallas_call(
        _mega_body,
        grid_spec=grid_spec,
        out_shape=jax.ShapeDtypeStruct((ELY, 2, C2, D), jnp.float32),
        compiler_params=pltpu.CompilerParams(
            dimension_semantics=("arbitrary", "arbitrary", "arbitrary"),
            collective_id=3,
        ),
    )(eids, x, dbuf_own, selp, W1, W2)


NCH = 8
CW = D // NCH


def _cb_xy_body(p_ref, o_ref, xrecv, yrecv, xs_sems, xr_sems, ys_sems, yr_sems):
    my_x = lax.axis_index("x")
    my_y = lax.axis_index("y")
    xpeer = (1 - my_x, my_y)
    ypeer = (my_x, 1 - my_y)

    barrier = pltpu.get_barrier_semaphore()
    for peer in (xpeer, ypeer):
        pl.semaphore_signal(
            barrier, inc=1, device_id=peer,
            device_id_type=pl.DeviceIdType.MESH,
        )
    pl.semaphore_wait(barrier, 2)

    x_rdmas = []
    for k in range(NCH):
        cs = pl.ds(k * CW, CW)
        r = pltpu.make_async_remote_copy(
            src_ref=p_ref.at[pl.ds((1 - my_x) * TS, TS), cs],
            dst_ref=xrecv.at[:, cs],
            send_sem=xs_sems.at[k],
            recv_sem=xr_sems.at[k],
            device_id=xpeer,
            device_id_type=pl.DeviceIdType.MESH,
        )
        r.start()
        x_rdmas.append(r)

    y_rdmas = []
    for k in range(NCH):
        cs = pl.ds(k * CW, CW)
        x_rdmas[k].wait_recv()
        o_ref[:, cs] = p_ref[pl.ds(my_x * TS, TS), cs] + xrecv[:, cs]
        r = pltpu.make_async_remote_copy(
            src_ref=o_ref.at[:, cs],
            dst_ref=yrecv.at[:, cs],
            send_sem=ys_sems.at[k],
            recv_sem=yr_sems.at[k],
            device_id=ypeer,
            device_id_type=pl.DeviceIdType.MESH,
        )
        r.start()
        y_rdmas.append(r)

    for k in range(NCH):
        cs = pl.ds(k * CW, CW)
        y_rdmas[k].wait_send()
        y_rdmas[k].wait_recv()
        o_ref[:, cs] = o_ref[:, cs] + yrecv[:, cs]

    for k in range(NCH):
        x_rdmas[k].wait_send()


def _combine_xy(partial):
    return pl.pallas_call(
        _cb_xy_body,
        out_shape=jax.ShapeDtypeStruct((TS, D), jnp.float32),
        in_specs=[pl.BlockSpec(memory_space=pltpu.VMEM)],
        out_specs=pl.BlockSpec(memory_space=pltpu.VMEM),
        scratch_shapes=[
            pltpu.VMEM((TS, D), jnp.float32),
            pltpu.VMEM((TS, D), jnp.float32),
            pltpu.SemaphoreType.DMA((NCH,)),
            pltpu.SemaphoreType.DMA((NCH,)),
            pltpu.SemaphoreType.DMA((NCH,)),
            pltpu.SemaphoreType.DMA((NCH,)),
        ],
        compiler_params=pltpu.CompilerParams(collective_id=1),
    )(partial)


def _route(x_shard, rt_full):
    gates = jnp.dot(x_shard, rt_full.T, precision=lax.Precision.HIGHEST)
    top2val, top2idx = lax.top_k(gates, K)
    w = jax.nn.softmax(top2val, axis=-1)
    flat_e = top2idx.reshape(-1)
    onehot = jax.nn.one_hot(flat_e, E, dtype=jnp.int32)
    pos = jnp.sum(jnp.cumsum(onehot, axis=0) * onehot, axis=1) - 1
    return top2idx, pos.reshape(TS, K), w


def kernel(x, router, W1, W2):
    my_x = lax.axis_index("x")
    my_y = lax.axis_index("y")
    e0 = my_x * EL + my_y * ELY

    rt_full = _router_ag(router.T)

    t2_mine, pos_mine, w_mine = _route(x, rt_full)
    meta_mine = jnp.concatenate(
        [
            t2_mine.astype(jnp.float32),
            pos_mine.astype(jnp.float32),
            w_mine,
            jnp.zeros((TS, 2), jnp.float32),
        ],
        axis=1,
    )
    meta_peer = _meta_exchange(meta_mine)
    t2_peer = meta_peer[:, 0:2].astype(jnp.int32)
    pos_peer = meta_peer[:, 2:4].astype(jnp.int32)
    w_peer = meta_peer[:, 4:6]

    def dest_ids(t2, pos):
        le = t2 - e0
        valid = (le >= 0) & (le < ELY) & (pos < C2)
        return jnp.where(valid, le * C2 + pos, ELY * C2)

    iota_d = jnp.arange(ELY * C2, dtype=jnp.int32)

    def sel_matrix(t2, pos):
        d2 = dest_ids(t2, pos)
        eq0 = iota_d[:, None] == d2[:, 0][None, :]
        eq1 = iota_d[:, None] == d2[:, 1][None, :]
        return eq0.astype(jnp.float32) + eq1.astype(jnp.float32)

    dbuf_own = jnp.dot(sel_matrix(t2_mine, pos_mine), x).reshape(ELY, C2, D)
    selp = sel_matrix(t2_peer, pos_peer).reshape(ELY, C2, TS)

    eids = my_y * ELY + jnp.arange(ELY, dtype=jnp.int32)
    ybuf = _mega_ffn(eids, x, dbuf_own, selp, W1, W2)

    cat = lambda a, b: jnp.concatenate([a, b], axis=0)
    swap = my_x == 1
    t2_all = jnp.where(swap, cat(t2_peer, t2_mine), cat(t2_mine, t2_peer))
    pos_all = jnp.where(
        swap, cat(pos_peer, pos_mine), cat(pos_mine, pos_peer)
    )
    w_all = jnp.where(swap, cat(w_peer, w_mine), cat(w_mine, w_peer))
    src_all = jnp.arange(T, dtype=jnp.int32) // TS
    sec_all = (src_all != my_x).astype(jnp.int32)

    le_all = t2_all - e0
    valid_all = (le_all >= 0) & (le_all < ELY) & (pos_all < C2)
    dest_all = jnp.where(
        valid_all,
        le_all * (2 * C2) + sec_all[:, None] * C2 + pos_all,
        ELY * 2 * C2,
    )
    iota_g = jnp.arange(ELY * 2 * C2, dtype=jnp.int32)
    eq0 = iota_g[:, None] == dest_all[:, 0][None, :]
    eq1 = iota_g[:, None] == dest_all[:, 1][None, :]
    selw = jnp.where(eq0, w_all[:, 0][None, :], 0.0) + jnp.where(
        eq1, w_all[:, 1][None, :], 0.0
    )

    partial = lax.dot_general(
        selw,
        ybuf.reshape(ELY * 2 * C2, D),
        (((0,), (0,)), ((), ())),
    )

    return _combine_xy(partial)


# device time: 144240 ns/iter; 1.3160x vs baseline; 1.3160x over previous
import jax
import jax.numpy as jnp
from jax import lax
from jax.experimental import pallas as pl
from jax.experimental.pallas import tpu as pltpu

T = 2048
TS = 1024
D = 1024
F = 4096
E = 16
EL = 8
ELY = 4
K = 2
C2 = 168
FT = 1024


def _rag_body(r_ref, rt_ref, send_sem, recv_sem):
    my_x = lax.axis_index("x")
    my_y = lax.axis_index("y")
    peer = (1 - my_x, my_y)

    barrier = pltpu.get_barrier_semaphore()
    pl.semaphore_signal(
        barrier, inc=1, device_id=peer, device_id_type=pl.DeviceIdType.MESH
    )
    pl.semaphore_wait(barrier, 1)

    rt_ref[pl.ds(my_x * EL, EL), :] = r_ref[...]
    rdma = pltpu.make_async_remote_copy(
        src_ref=r_ref,
        dst_ref=rt_ref.at[pl.ds(my_x * EL, EL), :],
        send_sem=send_sem,
        recv_sem=recv_sem,
        device_id=peer,
        device_id_type=pl.DeviceIdType.MESH,
    )
    rdma.start()
    rdma.wait()


def _router_ag(rt_shard):
    return pl.pallas_call(
        _rag_body,
        out_shape=jax.ShapeDtypeStruct((E, D), jnp.float32),
        in_specs=[pl.BlockSpec(memory_space=pltpu.VMEM)],
        out_specs=pl.BlockSpec(memory_space=pltpu.VMEM),
        scratch_shapes=[pltpu.SemaphoreType.DMA, pltpu.SemaphoreType.DMA],
        compiler_params=pltpu.CompilerParams(collective_id=0),
    )(rt_shard)


def _xmeta_body(m_ref, xb_ref, mout_ref, xout_ref, send_sems, recv_sems):
    my_x = lax.axis_index("x")
    my_y = lax.axis_index("y")
    peer = (1 - my_x, my_y)

    barrier = pltpu.get_barrier_semaphore()
    pl.semaphore_signal(
        barrier, inc=1, device_id=peer, device_id_type=pl.DeviceIdType.MESH
    )
    pl.semaphore_wait(barrier, 1)

    r1 = pltpu.make_async_remote_copy(
        src_ref=m_ref,
        dst_ref=mout_ref,
        send_sem=send_sems.at[0],
        recv_sem=recv_sems.at[0],
        device_id=peer,
        device_id_type=pl.DeviceIdType.MESH,
    )
    r2 = pltpu.make_async_remote_copy(
        src_ref=xb_ref,
        dst_ref=xout_ref,
        send_sem=send_sems.at[1],
        recv_sem=recv_sems.at[1],
        device_id=peer,
        device_id_type=pl.DeviceIdType.MESH,
    )
    r1.start()
    r2.start()
    r1.wait()
    r2.wait()


def _xmeta_exchange(meta_mine, xb):
    return pl.pallas_call(
        _xmeta_body,
        out_shape=[
            jax.ShapeDtypeStruct((TS, 8), jnp.float32),
            jax.ShapeDtypeStruct((TS, D), jnp.bfloat16),
        ],
        in_specs=[
            pl.BlockSpec(memory_space=pltpu.VMEM),
            pl.BlockSpec(memory_space=pltpu.VMEM),
        ],
        out_specs=[
            pl.BlockSpec(memory_space=pltpu.VMEM),
            pl.BlockSpec(memory_space=pltpu.VMEM),
        ],
        scratch_shapes=[
            pltpu.SemaphoreType.DMA((2,)),
            pltpu.SemaphoreType.DMA((2,)),
        ],
        compiler_params=pltpu.CompilerParams(collective_id=2),
    )(meta_mine, xb)


NF = F // FT


def _ffn_body(eids_ref, d_ref, w1_ref, w2_ref, y_ref):
    h = jnp.maximum(
        jnp.dot(d_ref[0], w1_ref[0], preferred_element_type=jnp.float32), 0.0
    )
    contrib = jnp.dot(h, w2_ref[0], preferred_element_type=jnp.float32)

    @pl.when(pl.program_id(1) == 0)
    def _():
        y_ref[0, :, :] = contrib

    @pl.when(pl.program_id(1) != 0)
    def _():
        y_ref[0, :, :] += contrib


def _expert_ffn(eids, dbuf, W1, W2):
    grid_spec = pltpu.PrefetchScalarGridSpec(
        num_scalar_prefetch=1,
        grid=(ELY, NF),
        in_specs=[
            pl.BlockSpec((1, 2 * C2, D), lambda e, f, eids: (e, 0, 0)),
            pl.BlockSpec((1, D, FT), lambda e, f, eids: (eids[e], 0, f)),
            pl.BlockSpec((1, FT, D), lambda e, f, eids: (eids[e], f, 0)),
        ],
        out_specs=pl.BlockSpec((1, 2 * C2, D), lambda e, f, eids: (e, 0, 0)),
    )
    return pl.pallas_call(
        _ffn_body,
        grid_spec=grid_spec,
        out_shape=jax.ShapeDtypeStruct((ELY, 2 * C2, D), jnp.float32),
        compiler_params=pltpu.CompilerParams(
            dimension_semantics=("arbitrary", "arbitrary")
        ),
    )(eids, dbuf, W1, W2)


NCH = 8
CW = D // NCH


def _cb_xy_body(
    p_ref, o_ref, xsend, xrecv, ysend, yrecv,
    xs_sems, xr_sems, ys_sems, yr_sems,
):
    my_x = lax.axis_index("x")
    my_y = lax.axis_index("y")
    xpeer = (1 - my_x, my_y)
    ypeer = (my_x, 1 - my_y)

    barrier = pltpu.get_barrier_semaphore()
    for peer in (xpeer, ypeer):
        pl.semaphore_signal(
            barrier, inc=1, device_id=peer,
            device_id_type=pl.DeviceIdType.MESH,
        )
    pl.semaphore_wait(barrier, 2)

    x_rdmas = []
    for k in range(NCH):
        cs = pl.ds(k * CW, CW)
        xsend[:, cs] = p_ref[pl.ds((1 - my_x) * TS, TS), cs].astype(
            jnp.bfloat16
        )
        r = pltpu.make_async_remote_copy(
            src_ref=xsend.at[:, cs],
            dst_ref=xrecv.at[:, cs],
            send_sem=xs_sems.at[k],
            recv_sem=xr_sems.at[k],
            device_id=xpeer,
            device_id_type=pl.DeviceIdType.MESH,
        )
        r.start()
        x_rdmas.append(r)

    y_rdmas = []
    for k in range(NCH):
        cs = pl.ds(k * CW, CW)
        x_rdmas[k].wait_recv()
        o_ref[:, cs] = p_ref[pl.ds(my_x * TS, TS), cs] + xrecv[
            :, cs
        ].astype(jnp.float32)
        ysend[:, cs] = o_ref[:, cs].astype(jnp.bfloat16)
        r = pltpu.make_async_remote_copy(
            src_ref=ysend.at[:, cs],
            dst_ref=yrecv.at[:, cs],
            send_sem=ys_sems.at[k],
            recv_sem=yr_sems.at[k],
            device_id=ypeer,
            device_id_type=pl.DeviceIdType.MESH,
        )
        r.start()
        y_rdmas.append(r)

    for k in range(NCH):
        cs = pl.ds(k * CW, CW)
        y_rdmas[k].wait_recv()
        o_ref[:, cs] = o_ref[:, cs] + yrecv[:, cs].astype(jnp.float32)

    for k in range(NCH):
        x_rdmas[k].wait_send()
        y_rdmas[k].wait_send()


def _combine_xy(partial):
    return pl.pallas_call(
        _cb_xy_body,
        out_shape=jax.ShapeDtypeStruct((TS, D), jnp.float32),
        in_specs=[pl.BlockSpec(memory_space=pltpu.VMEM)],
        out_specs=pl.BlockSpec(memory_space=pltpu.VMEM),
        scratch_shapes=[
            pltpu.VMEM((TS, D), jnp.bfloat16),
            pltpu.VMEM((TS, D), jnp.bfloat16),
            pltpu.VMEM((TS, D), jnp.bfloat16),
            pltpu.VMEM((TS, D), jnp.bfloat16),
            pltpu.SemaphoreType.DMA((NCH,)),
            pltpu.SemaphoreType.DMA((NCH,)),
            pltpu.SemaphoreType.DMA((NCH,)),
            pltpu.SemaphoreType.DMA((NCH,)),
        ],
        compiler_params=pltpu.CompilerParams(collective_id=1),
    )(partial)


def _route(x_shard, rt_full):
    gates = jnp.dot(x_shard, rt_full.T, precision=lax.Precision.HIGHEST)
    top2val, top2idx = lax.top_k(gates, K)
    w = jax.nn.softmax(top2val, axis=-1)
    flat_e = top2idx.reshape(-1)
    onehot = jax.nn.one_hot(flat_e, E, dtype=jnp.int32)
    pos = jnp.sum(jnp.cumsum(onehot, axis=0) * onehot, axis=1) - 1
    return top2idx, pos.reshape(TS, K), w


def kernel(x, router, W1, W2):
    my_x = lax.axis_index("x")
    my_y = lax.axis_index("y")
    e0 = my_x * EL + my_y * ELY

    rt_full = _router_ag(router.T)

    t2_mine, pos_mine, w_mine = _route(x, rt_full)
    meta_mine = jnp.concatenate(
        [
            t2_mine.astype(jnp.float32),
            pos_mine.astype(jnp.float32),
            w_mine,
            jnp.zeros((TS, 2), jnp.float32),
        ],
        axis=1,
    )
    meta_peer, xpeer_b = _xmeta_exchange(meta_mine, x.astype(jnp.bfloat16))
    t2_peer = meta_peer[:, 0:2].astype(jnp.int32)
    pos_peer = meta_peer[:, 2:4].astype(jnp.int32)
    w_peer = meta_peer[:, 4:6]

    def dest_ids(t2, pos):
        le = t2 - e0
        valid = (le >= 0) & (le < ELY) & (pos < C2)
        return jnp.where(valid, le * C2 + pos, ELY * C2)

    iota_d = jnp.arange(ELY * C2, dtype=jnp.int32)

    def sel_matrix(t2, pos):
        d2 = dest_ids(t2, pos)
        eq0 = iota_d[:, None] == d2[:, 0][None, :]
        eq1 = iota_d[:, None] == d2[:, 1][None, :]
        return eq0.astype(jnp.float32) + eq1.astype(jnp.float32)

    dbuf_own = jnp.dot(sel_matrix(t2_mine, pos_mine), x).reshape(ELY, C2, D)
    dbuf_peer = jnp.dot(
        sel_matrix(t2_peer, pos_peer), xpeer_b.astype(jnp.float32)
    ).reshape(ELY, C2, D)
    dbuf = jnp.concatenate([dbuf_own, dbuf_peer], axis=1)

    eids = my_y * ELY + jnp.arange(ELY, dtype=jnp.int32)
    ybuf = _expert_ffn(eids, dbuf, W1, W2)

    cat = lambda a, b: jnp.concatenate([a, b], axis=0)
    swap = my_x == 1
    t2_all = jnp.where(swap, cat(t2_peer, t2_mine), cat(t2_mine, t2_peer))
    pos_all = jnp.where(
        swap, cat(pos_peer, pos_mine), cat(pos_mine, pos_peer)
    )
    w_all = jnp.where(swap, cat(w_peer, w_mine), cat(w_mine, w_peer))
    src_all = jnp.arange(T, dtype=jnp.int32) // TS
    sec_all = (src_all != my_x).astype(jnp.int32)

    le_all = t2_all - e0
    valid_all = (le_all >= 0) & (le_all < ELY) & (pos_all < C2)
    dest_all = jnp.where(
        valid_all,
        le_all * (2 * C2) + sec_all[:, None] * C2 + pos_all,
        ELY * 2 * C2,
    )
    iota_g = jnp.arange(ELY * 2 * C2, dtype=jnp.int32)
    eq0 = iota_g[:, None] == dest_all[:, 0][None, :]
    eq1 = iota_g[:, None] == dest_all[:, 1][None, :]
    selw = jnp.where(eq0, w_all[:, 0][None, :], 0.0) + jnp.where(
        eq1, w_all[:, 1][None, :], 0.0
    )

    partial = lax.dot_general(
        selw,
        ybuf.reshape(ELY * 2 * C2, D),
        (((0,), (0,)), ((), ())),
    )

    return _combine_xy(partial)


# device time: 133833 ns/iter; 1.4183x vs baseline; 1.0778x over previous
import jax
import jax.numpy as jnp
from jax import lax
from jax.experimental import pallas as pl
from jax.experimental.pallas import tpu as pltpu

T = 2048
TS = 1024
D = 1024
F = 4096
E = 16
EL = 8
ELY = 4
K = 2
C2 = 168
FT = 1024


def _rag_body(r_ref, rt_ref, send_sem, recv_sem):
    my_x = lax.axis_index("x")
    my_y = lax.axis_index("y")
    peer = (1 - my_x, my_y)

    barrier = pltpu.get_barrier_semaphore()
    pl.semaphore_signal(
        barrier, inc=1, device_id=peer, device_id_type=pl.DeviceIdType.MESH
    )
    pl.semaphore_wait(barrier, 1)

    rt_ref[pl.ds(my_x * EL, EL), :] = r_ref[...]
    rdma = pltpu.make_async_remote_copy(
        src_ref=r_ref,
        dst_ref=rt_ref.at[pl.ds(my_x * EL, EL), :],
        send_sem=send_sem,
        recv_sem=recv_sem,
        device_id=peer,
        device_id_type=pl.DeviceIdType.MESH,
    )
    rdma.start()
    rdma.wait()


def _router_ag(rt_shard):
    return pl.pallas_call(
        _rag_body,
        out_shape=jax.ShapeDtypeStruct((E, D), jnp.float32),
        in_specs=[pl.BlockSpec(memory_space=pltpu.VMEM)],
        out_specs=pl.BlockSpec(memory_space=pltpu.VMEM),
        scratch_shapes=[pltpu.SemaphoreType.DMA, pltpu.SemaphoreType.DMA],
        compiler_params=pltpu.CompilerParams(collective_id=0),
    )(rt_shard)


def _xmeta_body(
    m_ref, xb_ref, selo_ref, x_ref, mout_ref, dbuf_ref,
    xpeer_buf, send_sems, recv_sems,
):
    my_x = lax.axis_index("x")
    my_y = lax.axis_index("y")
    peer = (1 - my_x, my_y)
    e0 = my_x * EL + my_y * ELY

    barrier = pltpu.get_barrier_semaphore()
    pl.semaphore_signal(
        barrier, inc=1, device_id=peer, device_id_type=pl.DeviceIdType.MESH
    )
    pl.semaphore_wait(barrier, 1)

    r1 = pltpu.make_async_remote_copy(
        src_ref=m_ref,
        dst_ref=mout_ref,
        send_sem=send_sems.at[0],
        recv_sem=recv_sems.at[0],
        device_id=peer,
        device_id_type=pl.DeviceIdType.MESH,
    )
    r2 = pltpu.make_async_remote_copy(
        src_ref=xb_ref,
        dst_ref=xpeer_buf,
        send_sem=send_sems.at[1],
        recv_sem=recv_sems.at[1],
        device_id=peer,
        device_id_type=pl.DeviceIdType.MESH,
    )
    r1.start()
    r2.start()

    dflat = jnp.dot(
        selo_ref[...], x_ref[...], preferred_element_type=jnp.float32
    )
    for e in range(ELY):
        dbuf_ref[e, 0:C2, :] = dflat[e * C2 : (e + 1) * C2, :]

    r1.wait_recv()
    mp = mout_ref[...]
    iota_r = lax.broadcasted_iota(jnp.int32, (ELY * C2, TS), 0)
    selp = jnp.zeros((ELY * C2, TS), jnp.float32)
    for k in range(K):
        t2 = mp[k : k + 1, :].astype(jnp.int32)
        ps = mp[K + k : K + k + 1, :].astype(jnp.int32)
        le = t2 - e0
        valid = (le >= 0) & (le < ELY) & (ps < C2)
        dest = jnp.where(valid, le * C2 + ps, ELY * C2)
        selp = selp + (iota_r == dest).astype(jnp.float32)

    r2.wait_recv()
    pflat = jnp.dot(
        selp,
        xpeer_buf[...].astype(jnp.float32),
        preferred_element_type=jnp.float32,
    )
    for e in range(ELY):
        dbuf_ref[e, C2 : 2 * C2, :] = pflat[e * C2 : (e + 1) * C2, :]

    r1.wait_send()
    r2.wait_send()


def _xmeta_exchange(meta_mine, xb, sel_own, x):
    return pl.pallas_call(
        _xmeta_body,
        out_shape=[
            jax.ShapeDtypeStruct((8, TS), jnp.float32),
            jax.ShapeDtypeStruct((ELY, 2 * C2, D), jnp.float32),
        ],
        in_specs=[pl.BlockSpec(memory_space=pltpu.VMEM)] * 4,
        out_specs=[pl.BlockSpec(memory_space=pltpu.VMEM)] * 2,
        scratch_shapes=[
            pltpu.VMEM((TS, D), jnp.bfloat16),
            pltpu.SemaphoreType.DMA((2,)),
            pltpu.SemaphoreType.DMA((2,)),
        ],
        compiler_params=pltpu.CompilerParams(collective_id=2),
    )(meta_mine, xb, sel_own, x)


NF = F // FT


def _ffn_body(eids_ref, d_ref, w1_ref, w2_ref, y_ref):
    h = jnp.maximum(
        jnp.dot(d_ref[0], w1_ref[0], preferred_element_type=jnp.float32), 0.0
    )
    contrib = jnp.dot(h, w2_ref[0], preferred_element_type=jnp.float32)

    @pl.when(pl.program_id(1) == 0)
    def _():
        y_ref[0, :, :] = contrib

    @pl.when(pl.program_id(1) != 0)
    def _():
        y_ref[0, :, :] += contrib


def _expert_ffn(eids, dbuf, W1, W2):
    grid_spec = pltpu.PrefetchScalarGridSpec(
        num_scalar_prefetch=1,
        grid=(ELY, NF),
        in_specs=[
            pl.BlockSpec((1, 2 * C2, D), lambda e, f, eids: (e, 0, 0)),
            pl.BlockSpec((1, D, FT), lambda e, f, eids: (eids[e], 0, f)),
            pl.BlockSpec((1, FT, D), lambda e, f, eids: (eids[e], f, 0)),
        ],
        out_specs=pl.BlockSpec((1, 2 * C2, D), lambda e, f, eids: (e, 0, 0)),
    )
    return pl.pallas_call(
        _ffn_body,
        grid_spec=grid_spec,
        out_shape=jax.ShapeDtypeStruct((ELY, 2 * C2, D), jnp.float32),
        compiler_params=pltpu.CompilerParams(
            dimension_semantics=("arbitrary", "arbitrary")
        ),
    )(eids, dbuf, W1, W2)


NCH = 8
CW = D // NCH


def _cb_xy_body(
    p_ref, o_ref, xsend, xrecv, ysend, yrecv,
    xs_sems, xr_sems, ys_sems, yr_sems,
):
    my_x = lax.axis_index("x")
    my_y = lax.axis_index("y")
    xpeer = (1 - my_x, my_y)
    ypeer = (my_x, 1 - my_y)

    barrier = pltpu.get_barrier_semaphore()
    for peer in (xpeer, ypeer):
        pl.semaphore_signal(
            barrier, inc=1, device_id=peer,
            device_id_type=pl.DeviceIdType.MESH,
        )
    pl.semaphore_wait(barrier, 2)

    x_rdmas = []
    for k in range(NCH):
        cs = pl.ds(k * CW, CW)
        xsend[:, cs] = p_ref[pl.ds((1 - my_x) * TS, TS), cs].astype(
            jnp.bfloat16
        )
        r = pltpu.make_async_remote_copy(
            src_ref=xsend.at[:, cs],
            dst_ref=xrecv.at[:, cs],
            send_sem=xs_sems.at[k],
            recv_sem=xr_sems.at[k],
            device_id=xpeer,
            device_id_type=pl.DeviceIdType.MESH,
        )
        r.start()
        x_rdmas.append(r)

    y_rdmas = []
    for k in range(NCH):
        cs = pl.ds(k * CW, CW)
        x_rdmas[k].wait_recv()
        o_ref[:, cs] = p_ref[pl.ds(my_x * TS, TS), cs] + xrecv[
            :, cs
        ].astype(jnp.float32)
        ysend[:, cs] = o_ref[:, cs].astype(jnp.bfloat16)
        r = pltpu.make_async_remote_copy(
            src_ref=ysend.at[:, cs],
            dst_ref=yrecv.at[:, cs],
            send_sem=ys_sems.at[k],
            recv_sem=yr_sems.at[k],
            device_id=ypeer,
            device_id_type=pl.DeviceIdType.MESH,
        )
        r.start()
        y_rdmas.append(r)

    for k in range(NCH):
        cs = pl.ds(k * CW, CW)
        y_rdmas[k].wait_recv()
        o_ref[:, cs] = o_ref[:, cs] + yrecv[:, cs].astype(jnp.float32)

    for k in range(NCH):
        x_rdmas[k].wait_send()
        y_rdmas[k].wait_send()


def _combine_xy(partial):
    return pl.pallas_call(
        _cb_xy_body,
        out_shape=jax.ShapeDtypeStruct((TS, D), jnp.float32),
        in_specs=[pl.BlockSpec(memory_space=pltpu.VMEM)],
        out_specs=pl.BlockSpec(memory_space=pltpu.VMEM),
        scratch_shapes=[
            pltpu.VMEM((TS, D), jnp.bfloat16),
            pltpu.VMEM((TS, D), jnp.bfloat16),
            pltpu.VMEM((TS, D), jnp.bfloat16),
            pltpu.VMEM((TS, D), jnp.bfloat16),
            pltpu.SemaphoreType.DMA((NCH,)),
            pltpu.SemaphoreType.DMA((NCH,)),
            pltpu.SemaphoreType.DMA((NCH,)),
            pltpu.SemaphoreType.DMA((NCH,)),
        ],
        compiler_params=pltpu.CompilerParams(collective_id=1),
    )(partial)


def _route(x_shard, rt_full):
    gates = jnp.dot(x_shard, rt_full.T, precision=lax.Precision.HIGHEST)
    top2val, top2idx = lax.top_k(gates, K)
    w = jax.nn.softmax(top2val, axis=-1)
    flat_e = top2idx.reshape(-1)
    onehot = jax.nn.one_hot(flat_e, E, dtype=jnp.int32)
    pos = jnp.sum(jnp.cumsum(onehot, axis=0) * onehot, axis=1) - 1
    return top2idx, pos.reshape(TS, K), w


def kernel(x, router, W1, W2):
    my_x = lax.axis_index("x")
    my_y = lax.axis_index("y")
    e0 = my_x * EL + my_y * ELY

    rt_full = _router_ag(router.T)

    t2_mine, pos_mine, w_mine = _route(x, rt_full)
    meta_mine = jnp.concatenate(
        [
            t2_mine.T.astype(jnp.float32),
            pos_mine.T.astype(jnp.float32),
            w_mine.T,
            jnp.zeros((2, TS), jnp.float32),
        ],
        axis=0,
    )

    iota_d = jnp.arange(ELY * C2, dtype=jnp.int32)

    def sel_matrix(t2, pos):
        le = t2 - e0
        valid = (le >= 0) & (le < ELY) & (pos < C2)
        d2 = jnp.where(valid, le * C2 + pos, ELY * C2)
        eq0 = iota_d[:, None] == d2[:, 0][None, :]
        eq1 = iota_d[:, None] == d2[:, 1][None, :]
        return eq0.astype(jnp.float32) + eq1.astype(jnp.float32)

    sel_own = sel_matrix(t2_mine, pos_mine)
    meta_peer, dbuf = _xmeta_exchange(
        meta_mine, x.astype(jnp.bfloat16), sel_own, x
    )
    t2_peer = meta_peer[0:2, :].T.astype(jnp.int32)
    pos_peer = meta_peer[2:4, :].T.astype(jnp.int32)
    w_peer = meta_peer[4:6, :].T

    eids = my_y * ELY + jnp.arange(ELY, dtype=jnp.int32)
    ybuf = _expert_ffn(eids, dbuf, W1, W2)

    cat = lambda a, b: jnp.concatenate([a, b], axis=0)
    swap = my_x == 1
    t2_all = jnp.where(swap, cat(t2_peer, t2_mine), cat(t2_mine, t2_peer))
    pos_all = jnp.where(
        swap, cat(pos_peer, pos_mine), cat(pos_mine, pos_peer)
    )
    w_all = jnp.where(swap, cat(w_peer, w_mine), cat(w_mine, w_peer))
    src_all = jnp.arange(T, dtype=jnp.int32) // TS
    sec_all = (src_all != my_x).astype(jnp.int32)

    le_all = t2_all - e0
    valid_all = (le_all >= 0) & (le_all < ELY) & (pos_all < C2)
    dest_all = jnp.where(
        valid_all,
        le_all * (2 * C2) + sec_all[:, None] * C2 + pos_all,
        ELY * 2 * C2,
    )
    iota_g = jnp.arange(ELY * 2 * C2, dtype=jnp.int32)
    eq0 = iota_g[:, None] == dest_all[:, 0][None, :]
    eq1 = iota_g[:, None] == dest_all[:, 1][None, :]
    selw = jnp.where(eq0, w_all[:, 0][None, :], 0.0) + jnp.where(
        eq1, w_all[:, 1][None, :], 0.0
    )

    partial = lax.dot_general(
        selw,
        ybuf.reshape(ELY * 2 * C2, D),
        (((0,), (0,)), ((), ())),
    )

    return _combine_xy(partial)
